# baseline (speedup 1.0000x reference)
import functools

import jax
import jax.numpy as jnp
from jax.experimental import pallas as pl
from jax.experimental.pallas import tpu as pltpu

_EPS = 1e-5


def _dcn_block_kernel(x_ref, w_ref, wom_ref, b_ref, o_ref,
                      col_ref, a1_ref, gt_ref,
                      *, H, W, KH, KW, C):
    HW = H * W
    KK = KH * KW
    ph, pw = KH // 2, KW // 2

    pidx = jax.lax.broadcasted_iota(jnp.int32, (1, HW), 1)
    h_idx = pidx // W
    w_idx = pidx % W

    def im2col(src):
        src_bf = src.astype(jnp.bfloat16)
        zero = jnp.bfloat16(0.0)
        for k in range(KK):
            dy = k // KW - ph
            dx = k % KW - pw
            d = dy * W + dx
            xs = src_bf if d == 0 else pltpu.roll(src_bf, shift=(-d) % HW, axis=1)
            valid = ((h_idx + dy >= 0) & (h_idx + dy < H) &
                     (w_idx + dx >= 0) & (w_idx + dx < W))
            xs = jnp.where(valid, xs, zero)
            col_ref[k * C:(k + 1) * C, :] = xs

    im2col(x_ref[0].astype(jnp.float32))
    a1 = jnp.dot(w_ref[0], col_ref[...],
                 preferred_element_type=jnp.float32) + b_ref[0]
    a1 = jnp.maximum(a1, 0.0)
    a1_ref[...] = a1.astype(jnp.bfloat16)

    z = jnp.dot(wom_ref[...], a1_ref[...],
                preferred_element_type=jnp.float32)
    om = jnp.zeros((32, HW), jnp.float32)
    for j in range(KK):
        dy = j // KW - ph
        dx = j % KW - pw
        d = dy * W + dx
        zj = z[32 * j:32 * (j + 1), :]
        zj = zj if d == 0 else pltpu.roll(zj, shift=(-d) % HW, axis=1)
        valid = ((h_idx + dy >= 0) & (h_idx + dy < H) &
                 (w_idx + dx >= 0) & (w_idx + dx < W))
        om = om + jnp.where(valid, zj, 0.0)

    h_f = h_idx.astype(jnp.float32)
    w_f = w_idx.astype(jnp.float32)
    qx_iota = jax.lax.broadcasted_iota(jnp.int32, (W, HW), 0)

    for k in range(KK):
        ky = k // KW
        kx = k % KW
        off_y = om[2 * k:2 * k + 1, :]
        off_x = om[2 * k + 1:2 * k + 2, :]
        msk = 2.0 / (1.0 + jnp.exp(-om[2 * KK + k:2 * KK + k + 1, :]))
        py = h_f + (ky - ph) + off_y
        px = w_f + (kx - pw) + off_x
        y0 = jnp.floor(py)
        x0 = jnp.floor(px)
        ly = py - y0
        lx = px - x0
        y0i = y0.astype(jnp.int32)
        x0i = x0.astype(jnp.int32)

        cw = (jnp.where(qx_iota == x0i, 1.0 - lx, 0.0) +
              jnp.where(qx_iota == x0i + 1, lx, 0.0))
        cwm = (cw * msk).astype(jnp.bfloat16)

        buf = k % 2
        for qy in range(H):
            wy = (jnp.where(y0i == qy, 1.0 - ly, 0.0) +
                  jnp.where(y0i == qy - 1, ly, 0.0))
            gt_ref[buf, qy * W:(qy + 1) * W, :] = wy.astype(jnp.bfloat16) * cwm

        samp = jnp.dot(a1_ref[...], gt_ref[buf],
                       preferred_element_type=jnp.float32)
        col_ref[k * C:(k + 1) * C, :] = samp.astype(jnp.bfloat16)

    a2 = jnp.dot(w_ref[1], col_ref[...],
                 preferred_element_type=jnp.float32) + b_ref[1]
    a2 = jnp.maximum(a2, 0.0)

    im2col(a2)
    out = jnp.dot(w_ref[2], col_ref[...],
                  preferred_element_type=jnp.float32) + b_ref[2]
    o_ref[0] = jnp.maximum(out, 0.0).astype(o_ref.dtype)


def kernel(x, w1, w2, w3, w_off, w_msk,
           g1, b1, m1, v1, g2, b2, m2, v2, g3, b3, m3, v3):
    N, C_in, H, W = x.shape
    F, _, KH, KW = w1.shape
    KK = KH * KW
    HW = H * W

    g = jnp.stack([g1, g2, g3])
    b = jnp.stack([b1, b2, b3])
    m = jnp.stack([m1, m2, m3])
    v = jnp.stack([v1, v2, v3])
    s = g * jax.lax.rsqrt(v + _EPS)
    wstk = jnp.stack([w1, w2, w3]) * s[:, :, None, None, None]
    w_all = (jnp.transpose(wstk, (0, 1, 3, 4, 2))
             .reshape(3, F, KK * F).astype(jnp.bfloat16))
    b_all = (b - m * s).reshape(3, F, 1)

    wom = jnp.concatenate([w_off, w_msk], axis=0)
    wom = jnp.pad(wom, ((0, 32 - 3 * KK), (0, 0), (0, 0), (0, 0)))
    womt = (jnp.transpose(wom, (2, 3, 0, 1))
            .reshape(KK * 32, F).astype(jnp.bfloat16))

    kern = functools.partial(_dcn_block_kernel, H=H, W=W, KH=KH, KW=KW, C=F)
    out = pl.pallas_call(
        kern,
        out_shape=jax.ShapeDtypeStruct((N, F, HW), x.dtype),
        grid=(N,),
        in_specs=[
            pl.BlockSpec((1, C_in, HW), lambda n: (n, 0, 0)),
            pl.BlockSpec((3, F, KK * F), lambda n: (0, 0, 0)),
            pl.BlockSpec((KK * 32, F), lambda n: (0, 0)),
            pl.BlockSpec((3, F, 1), lambda n: (0, 0, 0)),
        ],
        out_specs=pl.BlockSpec((1, F, HW), lambda n: (n, 0, 0)),
        scratch_shapes=[
            pltpu.VMEM((KK * F, HW), jnp.bfloat16),
            pltpu.VMEM((F, HW), jnp.bfloat16),
            pltpu.VMEM((2, HW, HW), jnp.bfloat16),
        ],
        compiler_params=pltpu.CompilerParams(
            dimension_semantics=("parallel",),
            vmem_limit_bytes=32 * 1024 * 1024,
        ),
    )(x.reshape(N, C_in, HW), w_all, womt, b_all)
    return out.reshape(N, F, H, W)


# speedup vs baseline: 1.0013x; 1.0013x over previous
import functools

import jax
import jax.numpy as jnp
from jax.experimental import pallas as pl
from jax.experimental.pallas import tpu as pltpu

_EPS = 1e-5


def _dcn_block_kernel(x_ref, w_ref, wom_ref, b_ref, o_ref,
                      col_ref, a1_ref, gt_ref,
                      *, H, W, KH, KW, C, PB):
    HW = H * W
    KK = KH * KW
    ph, pw = KH // 2, KW // 2

    pidx = jax.lax.broadcasted_iota(jnp.int32, (1, HW), 1)
    h_idx = pidx // W
    w_idx = pidx % W
    h_f = h_idx.astype(jnp.float32)
    w_f = w_idx.astype(jnp.float32)
    qx_iota = jax.lax.broadcasted_iota(jnp.int32, (W, HW), 0)

    def valid_mask(dy, dx):
        return ((h_idx + dy >= 0) & (h_idx + dy < H) &
                (w_idx + dx >= 0) & (w_idx + dx < W))

    def im2col(col, src):
        src_bf = src.astype(jnp.bfloat16)
        zero = jnp.bfloat16(0.0)
        for k in range(KK):
            dy = k // KW - ph
            dx = k % KW - pw
            d = dy * W + dx
            xs = src_bf if d == 0 else pltpu.roll(src_bf, shift=(-d) % HW, axis=1)
            col[k * C:(k + 1) * C, :] = jnp.where(valid_mask(dy, dx), xs, zero)

    def one_image(i):
        col = col_ref.at[i]
        a1r = a1_ref.at[i]

        im2col(col, x_ref[i].astype(jnp.float32))
        a1 = jnp.dot(w_ref[0], col[...],
                     preferred_element_type=jnp.float32) + b_ref[0]
        a1r[...] = jnp.maximum(a1, 0.0).astype(jnp.bfloat16)

        z = jnp.dot(wom_ref[...], a1r[...],
                    preferred_element_type=jnp.float32)
        om = jnp.zeros((32, HW), jnp.float32)
        for j in range(KK):
            dy = j // KW - ph
            dx = j % KW - pw
            d = dy * W + dx
            zj = z[32 * j:32 * (j + 1), :]
            zj = zj if d == 0 else pltpu.roll(zj, shift=(-d) % HW, axis=1)
            om = om + jnp.where(valid_mask(dy, dx), zj, 0.0)

        for k in range(KK):
            ky = k // KW
            kx = k % KW
            off_y = om[2 * k:2 * k + 1, :]
            off_x = om[2 * k + 1:2 * k + 2, :]
            msk = 2.0 / (1.0 + jnp.exp(-om[2 * KK + k:2 * KK + k + 1, :]))
            py = h_f + (ky - ph) + off_y
            px = w_f + (kx - pw) + off_x
            y0 = jnp.floor(py)
            x0 = jnp.floor(px)
            ly = py - y0
            lx = px - x0
            y0i = y0.astype(jnp.int32)
            x0i = x0.astype(jnp.int32)

            cw = (jnp.where(qx_iota == x0i, 1.0 - lx, 0.0) +
                  jnp.where(qx_iota == x0i + 1, lx, 0.0))
            cwm = (cw * msk).astype(jnp.bfloat16)

            buf = 2 * i + k % 2
            for qy in range(H):
                wy = (jnp.where(y0i == qy, 1.0 - ly, 0.0) +
                      jnp.where(y0i == qy - 1, ly, 0.0))
                gt_ref[buf, qy * W:(qy + 1) * W, :] = wy.astype(jnp.bfloat16) * cwm

            samp = jnp.dot(a1r[...], gt_ref[buf],
                           preferred_element_type=jnp.float32)
            col[k * C:(k + 1) * C, :] = samp.astype(jnp.bfloat16)

        a2 = jnp.dot(w_ref[1], col[...],
                     preferred_element_type=jnp.float32) + b_ref[1]
        a2 = jnp.maximum(a2, 0.0)

        im2col(col, a2)
        out = jnp.dot(w_ref[2], col[...],
                      preferred_element_type=jnp.float32) + b_ref[2]
        o_ref[i] = jnp.maximum(out, 0.0).astype(o_ref.dtype)

    for i in range(PB):
        one_image(i)


def kernel(x, w1, w2, w3, w_off, w_msk,
           g1, b1, m1, v1, g2, b2, m2, v2, g3, b3, m3, v3):
    N, C_in, H, W = x.shape
    F, _, KH, KW = w1.shape
    KK = KH * KW
    HW = H * W
    PB = 2

    g = jnp.stack([g1, g2, g3])
    b = jnp.stack([b1, b2, b3])
    m = jnp.stack([m1, m2, m3])
    v = jnp.stack([v1, v2, v3])
    s = g * jax.lax.rsqrt(v + _EPS)
    wstk = jnp.stack([w1, w2, w3]) * s[:, :, None, None, None]
    w_all = (jnp.transpose(wstk, (0, 1, 3, 4, 2))
             .reshape(3, F, KK * F).astype(jnp.bfloat16))
    b_all = (b - m * s).reshape(3, F, 1)

    wom = jnp.concatenate([w_off, w_msk], axis=0)
    wom = jnp.pad(wom, ((0, 32 - 3 * KK), (0, 0), (0, 0), (0, 0)))
    womt = (jnp.transpose(wom, (2, 3, 0, 1))
            .reshape(KK * 32, F).astype(jnp.bfloat16))

    kern = functools.partial(_dcn_block_kernel,
                             H=H, W=W, KH=KH, KW=KW, C=F, PB=PB)
    out = pl.pallas_call(
        kern,
        out_shape=jax.ShapeDtypeStruct((N, F, HW), x.dtype),
        grid=(N // PB,),
        in_specs=[
            pl.BlockSpec((PB, C_in, HW), lambda n: (n, 0, 0)),
            pl.BlockSpec((3, F, KK * F), lambda n: (0, 0, 0)),
            pl.BlockSpec((KK * 32, F), lambda n: (0, 0)),
            pl.BlockSpec((3, F, 1), lambda n: (0, 0, 0)),
        ],
        out_specs=pl.BlockSpec((PB, F, HW), lambda n: (n, 0, 0)),
        scratch_shapes=[
            pltpu.VMEM((PB, KK * F, HW), jnp.bfloat16),
            pltpu.VMEM((PB, F, HW), jnp.bfloat16),
            pltpu.VMEM((2 * PB, HW, HW), jnp.bfloat16),
        ],
        compiler_params=pltpu.CompilerParams(
            dimension_semantics=("parallel",),
            vmem_limit_bytes=64 * 1024 * 1024,
        ),
    )(x.reshape(N, C_in, HW), w_all, womt, b_all)
    return out.reshape(N, F, H, W)
